# baseline (speedup 1.0000x reference)
import jax
import jax.numpy as jnp
from jax import lax
from jax.experimental import pallas as pl
from jax.experimental.pallas import tpu as pltpu

N_NODES = 10000
N_EDGES = 320000
D = 128
N_CLS = 16
NG = 2

RB = 1000
NB = N_NODES // RB


def _pro_body(x_b, degs_b, degd_b, h_b, nd_b):
  ds_ = degs_b[0]
  dd = degd_b[0]
  ns = jnp.where(ds_ > 0, lax.rsqrt(jnp.maximum(ds_, 1.0)), 0.0)
  nd = jnp.where(dd > 0, lax.rsqrt(jnp.maximum(dd, 1.0)), 0.0)
  h_b[...] = (x_b[0] * ns)[None]
  nd_b[...] = nd[None]


_prologue = pl.pallas_call(
    _pro_body,
    grid=(NG, NB),
    in_specs=[
        pl.BlockSpec((1, RB, D), lambda g, i: (g, i, 0)),
        pl.BlockSpec((1, RB, 1), lambda g, i: (g, i, 0)),
        pl.BlockSpec((1, RB, 1), lambda g, i: (g, i, 0)),
    ],
    out_specs=[
        pl.BlockSpec((1, RB, D), lambda g, i: (g, i, 0)),
        pl.BlockSpec((1, RB, 1), lambda g, i: (g, i, 0)),
    ],
    out_shape=[
        jax.ShapeDtypeStruct((NG, N_NODES, D), jnp.float32),
        jax.ShapeDtypeStruct((NG, N_NODES, 1), jnp.float32),
    ],
)


def _epi_body(agg_b, nd_b, wconv_b, bconv_b, wcls_b, bcls_b, out_b, acc):
  g = pl.program_id(0)
  i = pl.program_id(1)

  a = agg_b[0] * nd_b[0]
  t = jnp.dot(a, wconv_b[...], preferred_element_type=jnp.float32)
  t = t + bconv_b[...]
  nr = jnp.sqrt(jnp.sum(t * t, axis=1, keepdims=True))
  t = t / jnp.maximum(nr, 1e-12)
  t = jax.nn.sigmoid(t)
  t = jnp.maximum(t, 0.0)
  colsum = jnp.sum(t, axis=0, keepdims=True)

  @pl.when(jnp.logical_and(g == 0, i == 0))
  def _():
    acc[...] = jnp.zeros((8, D), jnp.float32)

  gmask = lax.broadcasted_iota(jnp.int32, (8, D), 0) == g
  acc[...] += jnp.where(gmask, jnp.broadcast_to(colsum, (8, D)), 0.0)

  @pl.when(jnp.logical_and(g == NG - 1, i == NB - 1))
  def _():
    hg = acc[...] * (1.0 / N_NODES)
    o = jnp.dot(hg, wcls_b[...], preferred_element_type=jnp.float32)
    o = o + bcls_b[...]
    diff = o[0:1] - o[1:2] + 1e-6
    d = jnp.sqrt(jnp.sum(diff * diff))
    out_b[...] = jnp.full((8, 128), d, jnp.float32)


_epilogue = pl.pallas_call(
    _epi_body,
    grid=(NG, NB),
    in_specs=[
        pl.BlockSpec((1, RB, D), lambda g, i: (g, i, 0)),
        pl.BlockSpec((1, RB, 1), lambda g, i: (g, i, 0)),
        pl.BlockSpec((D, D), lambda g, i: (0, 0)),
        pl.BlockSpec((1, D), lambda g, i: (0, 0)),
        pl.BlockSpec((D, N_CLS), lambda g, i: (0, 0)),
        pl.BlockSpec((1, N_CLS), lambda g, i: (0, 0)),
    ],
    out_specs=pl.BlockSpec((8, 128), lambda g, i: (0, 0)),
    out_shape=jax.ShapeDtypeStruct((8, 128), jnp.float32),
    scratch_shapes=[pltpu.VMEM((8, D), jnp.float32)],
)


def kernel(g1_edge_index, g2_edge_index, in_feat1, in_feat2,
           W_conv, b_conv, W_cls, b_cls):
  e1 = g1_edge_index.astype(jnp.int32)
  e2 = g2_edge_index.astype(jnp.int32)

  deg_idx = jnp.concatenate([
      e1[0], e1[1] + N_NODES, e2[0] + 2 * N_NODES, e2[1] + 3 * N_NODES])
  degs_all = jax.ops.segment_sum(
      jnp.ones((4 * N_EDGES,), jnp.float32), deg_idx,
      num_segments=4 * N_NODES).reshape(4, N_NODES, 1)
  degs = degs_all[0::2]
  degd = degs_all[1::2]

  x = jnp.stack([in_feat1, in_feat2])
  h, nd = _prologue(x, degs, degd)

  src_all = jnp.concatenate([e1[0], e2[0] + N_NODES])
  dst_all = jnp.concatenate([e1[1], e2[1] + N_NODES])
  h_flat = h.reshape(NG * N_NODES, D)
  msgs = jnp.take(h_flat, src_all, axis=0)
  agg = jnp.zeros((NG * N_NODES, D), jnp.float32).at[dst_all].add(
      msgs).reshape(NG, N_NODES, D)

  out = _epilogue(agg, nd, W_conv, b_conv.reshape(1, D),
                  W_cls, b_cls.reshape(1, N_CLS))
  return out[0, 0].reshape(1)


# speedup vs baseline: 1.2427x; 1.2427x over previous
import jax
import jax.numpy as jnp
from jax import lax
from jax.experimental import pallas as pl
from jax.experimental.pallas import tpu as pltpu

N_NODES = 10000
N_EDGES = 320000
D = 128
N_CLS = 16
NG = 2

RB = 1000
NB = N_NODES // RB


def _pro_body(x_b, degs_b, degd_b, h_b, nd_b):
  ds_ = degs_b[0]
  dd = degd_b[0]
  ns = jnp.where(ds_ > 0, lax.rsqrt(jnp.maximum(ds_, 1.0)), 0.0)
  nd = jnp.where(dd > 0, lax.rsqrt(jnp.maximum(dd, 1.0)), 0.0)
  h_b[...] = (x_b[0] * ns)[None]
  nd_b[...] = nd[None]


_prologue = pl.pallas_call(
    _pro_body,
    grid=(NG, NB),
    in_specs=[
        pl.BlockSpec((1, RB, D), lambda g, i: (g, i, 0)),
        pl.BlockSpec((1, RB, 1), lambda g, i: (g, i, 0)),
        pl.BlockSpec((1, RB, 1), lambda g, i: (g, i, 0)),
    ],
    out_specs=[
        pl.BlockSpec((1, RB, D), lambda g, i: (g, i, 0)),
        pl.BlockSpec((1, RB, 1), lambda g, i: (g, i, 0)),
    ],
    out_shape=[
        jax.ShapeDtypeStruct((NG, N_NODES, D), jnp.float32),
        jax.ShapeDtypeStruct((NG, N_NODES, 1), jnp.float32),
    ],
)


def _epi_body(agg_b, nd_b, wconv_b, bconv_b, wcls_b, bcls_b, out_b, acc):
  g = pl.program_id(0)
  i = pl.program_id(1)

  a = agg_b[0] * nd_b[0]
  t = jnp.dot(a, wconv_b[...], preferred_element_type=jnp.float32)
  t = t + bconv_b[...]
  nr = jnp.sqrt(jnp.sum(t * t, axis=1, keepdims=True))
  t = t / jnp.maximum(nr, 1e-12)
  t = jax.nn.sigmoid(t)
  t = jnp.maximum(t, 0.0)
  colsum = jnp.sum(t, axis=0, keepdims=True)

  @pl.when(jnp.logical_and(g == 0, i == 0))
  def _():
    acc[...] = jnp.zeros((8, D), jnp.float32)

  gmask = lax.broadcasted_iota(jnp.int32, (8, D), 0) == g
  acc[...] += jnp.where(gmask, jnp.broadcast_to(colsum, (8, D)), 0.0)

  @pl.when(jnp.logical_and(g == NG - 1, i == NB - 1))
  def _():
    hg = acc[...] * (1.0 / N_NODES)
    o = jnp.dot(hg, wcls_b[...], preferred_element_type=jnp.float32)
    o = o + bcls_b[...]
    diff = o[0:1] - o[1:2] + 1e-6
    d = jnp.sqrt(jnp.sum(diff * diff))
    out_b[...] = jnp.full((8, 128), d, jnp.float32)


_epilogue = pl.pallas_call(
    _epi_body,
    grid=(NG, NB),
    in_specs=[
        pl.BlockSpec((1, RB, D), lambda g, i: (g, i, 0)),
        pl.BlockSpec((1, RB, 1), lambda g, i: (g, i, 0)),
        pl.BlockSpec((D, D), lambda g, i: (0, 0)),
        pl.BlockSpec((1, D), lambda g, i: (0, 0)),
        pl.BlockSpec((D, N_CLS), lambda g, i: (0, 0)),
        pl.BlockSpec((1, N_CLS), lambda g, i: (0, 0)),
    ],
    out_specs=pl.BlockSpec((8, 128), lambda g, i: (0, 0)),
    out_shape=jax.ShapeDtypeStruct((8, 128), jnp.float32),
    scratch_shapes=[pltpu.VMEM((8, D), jnp.float32)],
)


def kernel(g1_edge_index, g2_edge_index, in_feat1, in_feat2,
           W_conv, b_conv, W_cls, b_cls):
  e1 = g1_edge_index.astype(jnp.int32)
  e2 = g2_edge_index.astype(jnp.int32)

  deg_idx = jnp.concatenate([
      e1[0], e1[1] + N_NODES, e2[0] + 2 * N_NODES, e2[1] + 3 * N_NODES])
  degs_all = jax.ops.segment_sum(
      jnp.ones((4 * N_EDGES,), jnp.float32), deg_idx,
      num_segments=4 * N_NODES).reshape(4, N_NODES, 1)
  degs = degs_all[0::2]
  degd = degs_all[1::2]

  x = jnp.stack([in_feat1, in_feat2])
  h, nd = _prologue(x, degs, degd)

  zero = jnp.zeros((N_NODES, D), jnp.float32)
  agg = jnp.stack([
      zero.at[e1[1]].add(jnp.take(h[0], e1[0], axis=0)),
      zero.at[e2[1]].add(jnp.take(h[1], e2[0], axis=0)),
  ])

  out = _epilogue(agg, nd, W_conv, b_conv.reshape(1, D),
                  W_cls, b_cls.reshape(1, N_CLS))
  return out[0, 0].reshape(1)


# speedup vs baseline: 1.3449x; 1.0822x over previous
import jax
import jax.numpy as jnp
from jax import lax
from jax.experimental import pallas as pl
from jax.experimental.pallas import tpu as pltpu

N_NODES = 10000
N_EDGES = 320000
D = 128
N_CLS = 16
NG = 2

RB = 1000
NB = N_NODES // RB


def _pro_body(x_b, degs_b, degd_b, h_b, nd_b):
  ds_ = degs_b[0]
  dd = degd_b[0]
  ns = jnp.where(ds_ > 0, lax.rsqrt(jnp.maximum(ds_, 1.0)), 0.0)
  nd = jnp.where(dd > 0, lax.rsqrt(jnp.maximum(dd, 1.0)), 0.0)
  h_b[...] = (x_b[0] * ns)[None]
  nd_b[...] = nd[None]


_prologue = pl.pallas_call(
    _pro_body,
    grid=(NG, NB),
    in_specs=[
        pl.BlockSpec((1, RB, D), lambda g, i: (g, i, 0)),
        pl.BlockSpec((1, RB, 1), lambda g, i: (g, i, 0)),
        pl.BlockSpec((1, RB, 1), lambda g, i: (g, i, 0)),
    ],
    out_specs=[
        pl.BlockSpec((1, RB, D), lambda g, i: (g, i, 0)),
        pl.BlockSpec((1, RB, 1), lambda g, i: (g, i, 0)),
    ],
    out_shape=[
        jax.ShapeDtypeStruct((NG, N_NODES, D), jnp.float32),
        jax.ShapeDtypeStruct((NG, N_NODES, 1), jnp.float32),
    ],
)


def _epi_body(agg_b, nd_b, wconv_b, bconv_b, wcls_b, bcls_b, out_b, acc):
  g = pl.program_id(0)
  i = pl.program_id(1)

  a = agg_b[0] * nd_b[0]
  t = jnp.dot(a, wconv_b[...], preferred_element_type=jnp.float32)
  t = t + bconv_b[...]
  nr = jnp.sqrt(jnp.sum(t * t, axis=1, keepdims=True))
  t = t / jnp.maximum(nr, 1e-12)
  t = jax.nn.sigmoid(t)
  t = jnp.maximum(t, 0.0)
  colsum = jnp.sum(t, axis=0, keepdims=True)

  @pl.when(jnp.logical_and(g == 0, i == 0))
  def _():
    acc[...] = jnp.zeros((8, D), jnp.float32)

  gmask = lax.broadcasted_iota(jnp.int32, (8, D), 0) == g
  acc[...] += jnp.where(gmask, jnp.broadcast_to(colsum, (8, D)), 0.0)

  @pl.when(jnp.logical_and(g == NG - 1, i == NB - 1))
  def _():
    hg = acc[...] * (1.0 / N_NODES)
    o = jnp.dot(hg, wcls_b[...], preferred_element_type=jnp.float32)
    o = o + bcls_b[...]
    diff = o[0:1] - o[1:2] + 1e-6
    d = jnp.sqrt(jnp.sum(diff * diff))
    out_b[...] = jnp.full((8, 128), d, jnp.float32)


_epilogue = pl.pallas_call(
    _epi_body,
    grid=(NG, NB),
    in_specs=[
        pl.BlockSpec((1, RB, D), lambda g, i: (g, i, 0)),
        pl.BlockSpec((1, RB, 1), lambda g, i: (g, i, 0)),
        pl.BlockSpec((D, D), lambda g, i: (0, 0)),
        pl.BlockSpec((1, D), lambda g, i: (0, 0)),
        pl.BlockSpec((D, N_CLS), lambda g, i: (0, 0)),
        pl.BlockSpec((1, N_CLS), lambda g, i: (0, 0)),
    ],
    out_specs=pl.BlockSpec((8, 128), lambda g, i: (0, 0)),
    out_shape=jax.ShapeDtypeStruct((8, 128), jnp.float32),
    scratch_shapes=[pltpu.VMEM((8, D), jnp.float32)],
)


def kernel(g1_edge_index, g2_edge_index, in_feat1, in_feat2,
           W_conv, b_conv, W_cls, b_cls):
  e1 = g1_edge_index.astype(jnp.int32)
  e2 = g2_edge_index.astype(jnp.int32)

  ones2 = jnp.ones((2 * N_EDGES,), jnp.float32)
  degs = jax.ops.segment_sum(
      ones2, jnp.concatenate([e1[0], e2[0] + N_NODES]),
      num_segments=2 * N_NODES).reshape(NG, N_NODES, 1)
  degd = jax.ops.segment_sum(
      ones2, jnp.concatenate([e1[1], e2[1] + N_NODES]),
      num_segments=2 * N_NODES).reshape(NG, N_NODES, 1)

  x = jnp.stack([in_feat1, in_feat2])
  h, nd = _prologue(x, degs, degd)

  zero = jnp.zeros((N_NODES, D), jnp.float32)
  agg = jnp.stack([
      zero.at[e1[1]].add(jnp.take(h[0], e1[0], axis=0)),
      zero.at[e2[1]].add(jnp.take(h[1], e2[0], axis=0)),
  ])

  out = _epilogue(agg, nd, W_conv, b_conv.reshape(1, D),
                  W_cls, b_cls.reshape(1, N_CLS))
  return out[0, 0].reshape(1)


# speedup vs baseline: 1.3532x; 1.0062x over previous
import jax
import jax.numpy as jnp
from jax import lax
from jax.experimental import pallas as pl
from jax.experimental.pallas import tpu as pltpu

N_NODES = 10000
N_EDGES = 320000
D = 128
N_CLS = 16
NG = 2

RB = 2000
NB = N_NODES // RB


def _pro_body(x_b, degs_b, degd_b, h_b, nd_b):
  ds_ = degs_b[0]
  dd = degd_b[0]
  ns = jnp.where(ds_ > 0, lax.rsqrt(jnp.maximum(ds_, 1.0)), 0.0)
  nd = jnp.where(dd > 0, lax.rsqrt(jnp.maximum(dd, 1.0)), 0.0)
  h_b[...] = (x_b[0] * ns)[None]
  nd_b[...] = nd[None]


_prologue = pl.pallas_call(
    _pro_body,
    grid=(NG, 1),
    in_specs=[
        pl.BlockSpec((1, N_NODES, D), lambda g, i: (g, 0, 0)),
        pl.BlockSpec((1, N_NODES, 1), lambda g, i: (g, 0, 0)),
        pl.BlockSpec((1, N_NODES, 1), lambda g, i: (g, 0, 0)),
    ],
    out_specs=[
        pl.BlockSpec((1, N_NODES, D), lambda g, i: (g, 0, 0)),
        pl.BlockSpec((1, N_NODES, 1), lambda g, i: (g, 0, 0)),
    ],
    out_shape=[
        jax.ShapeDtypeStruct((NG, N_NODES, D), jnp.float32),
        jax.ShapeDtypeStruct((NG, N_NODES, 1), jnp.float32),
    ],
)


def _epi_body(agg_b, nd_b, wconv_b, bconv_b, wcls_b, bcls_b, out_b, acc):
  g = pl.program_id(0)
  i = pl.program_id(1)

  a = agg_b[0] * nd_b[0]
  t = jnp.dot(a, wconv_b[...], preferred_element_type=jnp.float32)
  t = t + bconv_b[...]
  nr = jnp.sqrt(jnp.sum(t * t, axis=1, keepdims=True))
  t = t / jnp.maximum(nr, 1e-12)
  t = jax.nn.sigmoid(t)
  t = jnp.maximum(t, 0.0)
  colsum = jnp.sum(t, axis=0, keepdims=True)

  @pl.when(jnp.logical_and(g == 0, i == 0))
  def _():
    acc[...] = jnp.zeros((8, D), jnp.float32)

  gmask = lax.broadcasted_iota(jnp.int32, (8, D), 0) == g
  acc[...] += jnp.where(gmask, jnp.broadcast_to(colsum, (8, D)), 0.0)

  @pl.when(jnp.logical_and(g == NG - 1, i == NB - 1))
  def _():
    hg = acc[...] * (1.0 / N_NODES)
    o = jnp.dot(hg, wcls_b[...], preferred_element_type=jnp.float32)
    o = o + bcls_b[...]
    diff = o[0:1] - o[1:2] + 1e-6
    d = jnp.sqrt(jnp.sum(diff * diff))
    out_b[...] = jnp.full((8, 128), d, jnp.float32)


_epilogue = pl.pallas_call(
    _epi_body,
    grid=(NG, NB),
    in_specs=[
        pl.BlockSpec((1, RB, D), lambda g, i: (g, i, 0)),
        pl.BlockSpec((1, RB, 1), lambda g, i: (g, i, 0)),
        pl.BlockSpec((D, D), lambda g, i: (0, 0)),
        pl.BlockSpec((1, D), lambda g, i: (0, 0)),
        pl.BlockSpec((D, N_CLS), lambda g, i: (0, 0)),
        pl.BlockSpec((1, N_CLS), lambda g, i: (0, 0)),
    ],
    out_specs=pl.BlockSpec((8, 128), lambda g, i: (0, 0)),
    out_shape=jax.ShapeDtypeStruct((8, 128), jnp.float32),
    scratch_shapes=[pltpu.VMEM((8, D), jnp.float32)],
)


def kernel(g1_edge_index, g2_edge_index, in_feat1, in_feat2,
           W_conv, b_conv, W_cls, b_cls):
  e1 = g1_edge_index.astype(jnp.int32)
  e2 = g2_edge_index.astype(jnp.int32)

  ones2 = jnp.ones((2 * N_EDGES,), jnp.float32)
  degs = jax.ops.segment_sum(
      ones2, jnp.concatenate([e1[0], e2[0] + N_NODES]),
      num_segments=2 * N_NODES).reshape(NG, N_NODES, 1)
  degd = jax.ops.segment_sum(
      ones2, jnp.concatenate([e1[1], e2[1] + N_NODES]),
      num_segments=2 * N_NODES).reshape(NG, N_NODES, 1)

  x = jnp.stack([in_feat1, in_feat2])
  h, nd = _prologue(x, degs, degd)

  zero = jnp.zeros((N_NODES, D), jnp.float32)
  agg = jnp.stack([
      zero.at[e1[1]].add(jnp.take(h[0], e1[0], axis=0)),
      zero.at[e2[1]].add(jnp.take(h[1], e2[0], axis=0)),
  ])

  out = _epilogue(agg, nd, W_conv, b_conv.reshape(1, D),
                  W_cls, b_cls.reshape(1, N_CLS))
  return out[0, 0].reshape(1)
